# baseline (speedup 1.0000x reference)
import functools

import jax
import jax.numpy as jnp
from jax import lax
from jax.experimental import pallas as pl
from jax.experimental.pallas import tpu as pltpu
from jax.experimental.pallas import tpu_sc as plsc

_NC = 2
_NS = 16
_NW = _NC * _NS

_TR = 8
_LANES = 16
_UNROLL = 2
_NBUF = 3


def _sc_add(B, seq_len, D):
    rows_per_w = seq_len // _NW
    steps = rows_per_w // _TR
    tile = _TR * D
    n_iter = tile // (_LANES * _UNROLL)
    groups = steps // _NBUF
    tail = steps - groups * _NBUF

    mesh = plsc.VectorSubcoreMesh(core_axis_name="c", subcore_axis_name="s")

    scratch = (
        [pltpu.VMEM((_TR, D), jnp.float32) for _ in range(_NBUF)]
        + [pltpu.VMEM((B, _TR, D), jnp.float32) for _ in range(_NBUF)]
        + [pltpu.SemaphoreType.DMA for _ in range(_NBUF)]
        + [pltpu.SemaphoreType.DMA for _ in range(_NBUF)]
        + [pltpu.SemaphoreType.DMA for _ in range(_NBUF)]
    )

    @functools.partial(
        pl.kernel,
        mesh=mesh,
        out_type=jax.ShapeDtypeStruct((B, seq_len, D), jnp.float32),
        scratch_types=scratch,
        compiler_params=pltpu.CompilerParams(use_tc_tiling_on_sc=True),
    )
    def run(x_hbm, pe_hbm, out_hbm, *s):
        k = _NBUF
        pe_v = s[0:k]
        x_v = s[k:2 * k]
        pe_sem = s[2 * k:3 * k]
        ld_sem = s[3 * k:4 * k]
        st_sem = s[4 * k:5 * k]

        wid = lax.axis_index("s") * _NC + lax.axis_index("c")
        base = wid * rows_per_w

        def pe_copy(t, q):
            return pltpu.make_async_copy(
                pe_hbm.at[pl.ds(base + t * _TR, _TR)], pe_v[q], pe_sem[q])

        def ld_copy(t, q):
            return pltpu.make_async_copy(
                x_hbm.at[:, pl.ds(base + t * _TR, _TR)], x_v[q], ld_sem[q])

        def st_copy(t, q):
            return pltpu.make_async_copy(
                x_v[q], out_hbm.at[:, pl.ds(base + t * _TR, _TR)], st_sem[q])

        pe_copy(0, 0).start()
        ld_copy(0, 0).start()

        def halfstep(t, q):
            nq = (q + 1) % _NBUF

            @pl.when(t + 1 < steps)
            def _():
                pe_copy(t + 1, nq).start()

                @pl.when(t >= 2)
                def _():
                    st_copy(t - 2, nq).wait()

                ld_copy(t + 1, nq).start()

            pe_copy(t, q).wait()
            ld_copy(t, q).wait()

            def add_body(j, c):
                for u in range(_UNROLL):
                    flat = (j * _UNROLL + u) * _LANES
                    r = flat // D
                    sl = pl.ds(flat % D, _LANES)
                    pv = pe_v[q][r, sl]
                    for b in range(B):
                        x_v[q][b, r, sl] = x_v[q][b, r, sl] + pv
                return c

            lax.fori_loop(0, n_iter, add_body, 0, unroll=False)

            st_copy(t, q).start()

        def outer(g, carry):
            for h in range(_NBUF):
                halfstep(g * _NBUF + h, h)
            return carry

        lax.fori_loop(0, groups, outer, 0, unroll=False)
        for h in range(tail):
            halfstep(groups * _NBUF + h, h)

        for t in range(steps - _NBUF, steps):
            st_copy(t, t % _NBUF).wait()

    return run


def kernel(x, pe_table):
    B, S, D = x.shape
    seq_len = min(S, pe_table.shape[0])
    return _sc_add(B, seq_len, D)(x[:, :seq_len, :], pe_table[:seq_len])


# speedup vs baseline: 1.5196x; 1.5196x over previous
import functools

import jax
import jax.numpy as jnp
from jax import lax
from jax.experimental import pallas as pl
from jax.experimental.pallas import tpu as pltpu
from jax.experimental.pallas import tpu_sc as plsc

_NC = 2
_NS = 16
_NW = _NC * _NS

_TR = 8
_LANES = 16
_UNROLL = 2
_NBUF = 3


def _sc_add_into_full(B, total_rows, D, row_off=0):
    rows_per_w = (total_rows - row_off) // _NW
    steps = rows_per_w // _TR
    tile = _TR * D
    n_iter = tile // (_LANES * _UNROLL)
    groups = steps // _NBUF
    tail = steps - groups * _NBUF

    mesh = plsc.VectorSubcoreMesh(core_axis_name="c", subcore_axis_name="s")

    scratch = (
        [pltpu.VMEM((_TR, D), jnp.float32) for _ in range(_NBUF)]
        + [pltpu.VMEM((_TR, D), jnp.float32) for _ in range(_NBUF * B)]
        + [pltpu.SemaphoreType.DMA for _ in range(_NBUF)]
        + [pltpu.SemaphoreType.DMA for _ in range(_NBUF * B)]
        + [pltpu.SemaphoreType.DMA for _ in range(_NBUF * B)]
    )

    @functools.partial(
        pl.kernel,
        mesh=mesh,
        out_type=jax.ShapeDtypeStruct((B, total_rows, D), jnp.float32),
        scratch_types=scratch,
        compiler_params=pltpu.CompilerParams(use_tc_tiling_on_sc=True),
    )
    def run(x_hbm, pe_hbm, out_hbm, *s):
        k = _NBUF
        pe_v = s[0:k]
        x_v = tuple(s[k + q * B:k + (q + 1) * B] for q in range(k))
        o = k + k * B
        pe_sem = s[o:o + k]
        ld_sem = tuple(s[o + k + q * B:o + k + (q + 1) * B] for q in range(k))
        o2 = o + k + k * B
        st_sem = tuple(s[o2 + q * B:o2 + (q + 1) * B] for q in range(k))

        wid = lax.axis_index("s") * _NC + lax.axis_index("c")
        base = wid * rows_per_w

        def pe_copy(t, q):
            return pltpu.make_async_copy(
                pe_hbm.at[pl.ds(row_off + base + t * _TR, _TR)], pe_v[q],
                pe_sem[q])

        def ld_copy(t, q, b):
            return pltpu.make_async_copy(
                x_hbm.at[b, pl.ds(row_off + base + t * _TR, _TR)], x_v[q][b],
                ld_sem[q][b])

        def st_copy(t, q, b):
            return pltpu.make_async_copy(
                x_v[q][b], out_hbm.at[b, pl.ds(row_off + base + t * _TR, _TR)],
                st_sem[q][b])

        pe_copy(0, 0).start()
        for b in range(B):
            ld_copy(0, 0, b).start()

        def halfstep(t, q):
            nq = (q + 1) % _NBUF

            @pl.when(t + 1 < steps)
            def _():
                pe_copy(t + 1, nq).start()

            for b in range(B):
                @pl.when(t + 1 < steps)
                def _():
                    @pl.when(t >= 2)
                    def _():
                        st_copy(t - 2, nq, b).wait()
                    ld_copy(t + 1, nq, b).start()

            pe_copy(t, q).wait()
            for b in range(B):
                ld_copy(t, q, b).wait()

            def add_body(j, c):
                for u in range(_UNROLL):
                    flat = (j * _UNROLL + u) * _LANES
                    r = flat // D
                    sl = pl.ds(flat % D, _LANES)
                    pv = pe_v[q][r, sl]
                    for b in range(B):
                        x_v[q][b][r, sl] = x_v[q][b][r, sl] + pv
                return c

            lax.fori_loop(0, n_iter, add_body, 0, unroll=False)

            for b in range(B):
                st_copy(t, q, b).start()

        def outer(g, carry):
            for h in range(_NBUF):
                halfstep(g * _NBUF + h, h)
            return carry

        lax.fori_loop(0, groups, outer, 0, unroll=False)
        for h in range(tail):
            halfstep(groups * _NBUF + h, h)

        for t in range(steps - _NBUF, steps):
            for b in range(B):
                st_copy(t, t % _NBUF, b).wait()

    return run


def _tc_body(x_ref, pe_ref, acc_ref, o_ref):
    del acc_ref
    o_ref[...] = x_ref[...] + pe_ref[...]


def kernel(x, pe_table):
    B, S, D = x.shape
    seq_len = min(S, pe_table.shape[0])
    xs = x[:, :seq_len, :]
    split = seq_len // 2
    BS = 512

    sc_full = _sc_add_into_full(B, seq_len, D, split)(xs, pe_table)

    return pl.pallas_call(
        _tc_body,
        grid=(split // BS, B),
        in_specs=[
            pl.BlockSpec((1, BS, D), lambda i, b: (b, i, 0)),
            pl.BlockSpec((BS, D), lambda i, b: (i, 0)),
            pl.BlockSpec(memory_space=pl.ANY),
        ],
        out_specs=pl.BlockSpec((1, BS, D), lambda i, b: (b, i, 0)),
        out_shape=jax.ShapeDtypeStruct((B, seq_len, D), x.dtype),
        input_output_aliases={2: 0},
    )(xs, pe_table, sc_full)


# speedup vs baseline: 1.5518x; 1.0212x over previous
import functools

import jax
import jax.numpy as jnp
from jax import lax
from jax.experimental import pallas as pl
from jax.experimental.pallas import tpu as pltpu
from jax.experimental.pallas import tpu_sc as plsc

_NC = 2
_NS = 16
_NW = _NC * _NS

_TR = 8
_LANES = 16
_UNROLL = 4
_NBUF = 3


def _sc_add(B, seq_len, D):
    rows_per_w = seq_len // _NW
    steps = rows_per_w // _TR
    tile = _TR * D
    n_iter = tile // (_LANES * _UNROLL)
    groups = steps // _NBUF
    tail = steps - groups * _NBUF

    mesh = plsc.VectorSubcoreMesh(core_axis_name="c", subcore_axis_name="s")

    scratch = (
        [pltpu.VMEM((_TR, D), jnp.float32) for _ in range(_NBUF)]
        + [pltpu.VMEM((_TR, D), jnp.float32) for _ in range(_NBUF * B)]
        + [pltpu.SemaphoreType.DMA for _ in range(_NBUF)]
        + [pltpu.SemaphoreType.DMA for _ in range(_NBUF * B)]
        + [pltpu.SemaphoreType.DMA for _ in range(_NBUF * B)]
    )

    @functools.partial(
        pl.kernel,
        mesh=mesh,
        out_type=jax.ShapeDtypeStruct((B, seq_len, D), jnp.float32),
        scratch_types=scratch,
        compiler_params=pltpu.CompilerParams(use_tc_tiling_on_sc=True),
    )
    def run(x_hbm, pe_hbm, out_hbm, *s):
        k = _NBUF
        pe_v = s[0:k]
        x_v = tuple(s[k + q * B:k + (q + 1) * B] for q in range(k))
        o = k + k * B
        pe_sem = s[o:o + k]
        ld_sem = tuple(s[o + k + q * B:o + k + (q + 1) * B] for q in range(k))
        o2 = o + k + k * B
        st_sem = tuple(s[o2 + q * B:o2 + (q + 1) * B] for q in range(k))

        wid = lax.axis_index("s") * _NC + lax.axis_index("c")
        base = wid * rows_per_w

        def pe_copy(t, q):
            return pltpu.make_async_copy(
                pe_hbm.at[pl.ds(base + t * _TR, _TR)], pe_v[q], pe_sem[q])

        def ld_copy(t, q, b):
            return pltpu.make_async_copy(
                x_hbm.at[b, pl.ds(base + t * _TR, _TR)], x_v[q][b],
                ld_sem[q][b])

        def st_copy(t, q, b):
            return pltpu.make_async_copy(
                x_v[q][b], out_hbm.at[b, pl.ds(base + t * _TR, _TR)],
                st_sem[q][b])

        pe_copy(0, 0).start()
        for b in range(B):
            ld_copy(0, 0, b).start()

        def halfstep(t, q):
            nq = (q + 1) % _NBUF

            @pl.when(t + 1 < steps)
            def _():
                pe_copy(t + 1, nq).start()

            for b in range(B):
                @pl.when(t + 1 < steps)
                def _():
                    @pl.when(t >= 2)
                    def _():
                        st_copy(t - 2, nq, b).wait()
                    ld_copy(t + 1, nq, b).start()

            pe_copy(t, q).wait()
            for b in range(B):
                ld_copy(t, q, b).wait()

            def add_body(j, c):
                for u in range(_UNROLL):
                    flat = (j * _UNROLL + u) * _LANES
                    r = flat // D
                    sl = pl.ds(flat % D, _LANES)
                    pv = pe_v[q][r, sl]
                    for b in range(B):
                        x_v[q][b][r, sl] = x_v[q][b][r, sl] + pv
                return c

            lax.fori_loop(0, n_iter, add_body, 0, unroll=False)

            for b in range(B):
                st_copy(t, q, b).start()

        def outer(g, carry):
            for h in range(_NBUF):
                halfstep(g * _NBUF + h, h)
            return carry

        lax.fori_loop(0, groups, outer, 0, unroll=False)
        for h in range(tail):
            halfstep(groups * _NBUF + h, h)

        for t in range(steps - _NBUF, steps):
            for b in range(B):
                st_copy(t, t % _NBUF, b).wait()

    return run


def kernel(x, pe_table):
    B, S, D = x.shape
    seq_len = min(S, pe_table.shape[0])
    return _sc_add(B, seq_len, D)(x[:, :seq_len, :], pe_table[:seq_len])
